# Initial kernel scaffold; baseline (speedup 1.0000x reference)
import jax
import jax.numpy as jnp
from jax.experimental import pallas as pl


def kernel(entity_emb, edge_index, edge_type, comp1, basis1, root1, bias1, comp2, basis2, root2, bias2):
    raise NotImplementedError("write your pallas kernel here")


# speedup vs baseline: 9.9311x; 9.9311x over previous
import functools

import jax
import jax.numpy as jnp
from jax import lax
from jax.experimental import pallas as pl
from jax.experimental.pallas import tpu as pltpu
from jax.experimental.pallas import tpu_sc as plsc

_N = 10000
_NP = 10240
_E = 160000
_R = 474
_RP = 480
_B = 30
_D1 = 200
_D2 = 100
_D1P = 256
_D2P = 128
_NF = _D1P * _D2P
_T = 128
_NB = 1728
_EP = _NB * _T
_NCH32 = _EP // (32 * _T)
_NCH16 = _EP // (16 * _T)
_KEYS = _N * _R
_C = 1_185_024
_CS = 1_310_720
_ZH = _CS // 16
_PAD_KEY = 1 << 30

@functools.cache
def _mesh():
    return plsc.VectorSubcoreMesh(core_axis_name="c", subcore_axis_name="s",
                                  num_cores=2, num_subcores=16)


def _coef_call(keys3, zh, zi):
    @functools.partial(
        pl.kernel,
        mesh=_mesh(),
        out_type=jax.ShapeDtypeStruct((2, 16, _NCH16, _T), jnp.float32),
        scratch_types=[
            pltpu.VMEM((_NCH16, _T), jnp.int32),
            pltpu.VMEM((_NCH16, _T), jnp.int32),
            pltpu.VMEM((_NCH16, _T), jnp.float32),
            pltpu.VMEM((_T,), jnp.float32),
            pltpu.VMEM((_T,), jnp.float32),
            pltpu.VMEM_SHARED((_CS,), jnp.float32),
        ],
    )
    def _coef_kernel(keys_hbm, zh_hbm, zi_hbm, coef_hbm,
                     keys_v, idx_v, inv_v, cnt_v, ones_v, hist_s):
        c = lax.axis_index("c")
        s = lax.axis_index("s")
        pltpu.sync_copy(keys_hbm.at[s], keys_v)
        pltpu.sync_copy(zi_hbm, inv_v)
        for k in range(_T // 16):
            ones_v[pl.ds(k * 16, 16)] = jnp.ones((16,), jnp.float32)
        for p in range(2):
            lo = (2 * p + c) * _C
            pltpu.sync_copy(zh_hbm, hist_s.at[pl.ds(s * _ZH, _ZH)])
            plsc.subcore_barrier()

            def _scatter(j, carry):
                for k in range(_T // 16):
                    key16 = keys_v[j, pl.ds(k * 16, 16)]
                    d = key16 - lo
                    inr = (d >= 0) & (d < _C)
                    idx_v[j, pl.ds(k * 16, 16)] = jnp.where(inr, d, _C)
                pltpu.sync_copy(ones_v, hist_s.at[idx_v.at[j]], add=True)
                return carry

            lax.fori_loop(0, _NCH16, _scatter, 0)
            plsc.subcore_barrier()

            def _gather(j, carry):
                pltpu.sync_copy(hist_s.at[idx_v.at[j]], cnt_v)
                for k in range(_T // 16):
                    key16 = keys_v[j, pl.ds(k * 16, 16)]
                    d = key16 - lo
                    inr = (d >= 0) & (d < _C)
                    cnt16 = cnt_v[pl.ds(k * 16, 16)]
                    add = jnp.where(inr, 1.0 / cnt16, 0.0)
                    sl = pl.ds(k * 16, 16)
                    inv_v[j, sl] = inv_v[j, sl] + add
                return carry

            lax.fori_loop(0, _NCH16, _gather, 0)
            plsc.subcore_barrier()
        pltpu.sync_copy(inv_v, coef_hbm.at[c, s])


    return _coef_kernel(keys3, zh, zi)

def _gather_call(x, srcp3, d):
    rows_pt = _NCH32 * _T

    @functools.partial(
        pl.kernel,
        mesh=_mesh(),
        out_type=jax.ShapeDtypeStruct((_EP, d), jnp.float32),
        scratch_types=[
            pltpu.VMEM((_NCH32, _T), jnp.int32),
            pltpu.VMEM((_T, d), jnp.float32),
            pltpu.VMEM((_T, d), jnp.float32),
            pltpu.SemaphoreType.DMA,
            pltpu.SemaphoreType.DMA,
        ],
    )
    def _g(x_hbm, srcp_hbm, xg_hbm, idx_v, r0, r1, sem0, sem1):
        c = lax.axis_index("c")
        s = lax.axis_index("s")
        w = s * 2 + c
        base = w * rows_pt
        pltpu.sync_copy(srcp_hbm.at[w], idx_v)
        bufs = (r0, r1)
        sems = (sem0, sem1)
        descs = [None, None]
        descs[0] = pltpu.async_copy(x_hbm.at[idx_v.at[0]], r0, sem0)
        for j in range(1, _NCH32 + 1):
            if j < _NCH32:
                descs[j % 2] = pltpu.async_copy(
                    x_hbm.at[idx_v.at[j]], bufs[j % 2], sems[j % 2])
            descs[(j - 1) % 2].wait()
            pltpu.sync_copy(bufs[(j - 1) % 2],
                            xg_hbm.at[pl.ds(base + (j - 1) * _T, _T)])

    return _g(x, srcp3)


def _scatter_call(msg, dstp3, zrows, d):
    rows_pt = _NCH32 * _T
    npt = _NP // 16

    @functools.partial(
        pl.kernel,
        mesh=_mesh(),
        out_type=jax.ShapeDtypeStruct((2, _NP, d), jnp.float32),
        scratch_types=[
            pltpu.VMEM((_NCH32, _T), jnp.int32),
            pltpu.VMEM((_T, d), jnp.float32),
            pltpu.VMEM((_T, d), jnp.float32),
            pltpu.VMEM_SHARED((_NP, d), jnp.float32),
            pltpu.SemaphoreType.DMA,
            pltpu.SemaphoreType.DMA,
        ],
    )
    def _s(msg_hbm, dstp_hbm, zr_hbm, out_hbm, idx_v, m0, m1, acc_s, sem0, sem1):
        c = lax.axis_index("c")
        s = lax.axis_index("s")
        w = s * 2 + c
        base = w * rows_pt
        pltpu.sync_copy(dstp_hbm.at[w], idx_v)
        pltpu.sync_copy(zr_hbm, acc_s.at[pl.ds(s * npt, npt)])
        plsc.subcore_barrier()
        bufs = (m0, m1)
        sems = (sem0, sem1)
        descs = [None, None]
        descs[0] = pltpu.async_copy(msg_hbm.at[pl.ds(base, _T)], m0, sem0)
        for j in range(1, _NCH32 + 1):
            if j < _NCH32:
                descs[j % 2] = pltpu.async_copy(
                    msg_hbm.at[pl.ds(base + j * _T, _T)], bufs[j % 2], sems[j % 2])
            descs[(j - 1) % 2].wait()
            pltpu.sync_copy(bufs[(j - 1) % 2], acc_s.at[idx_v.at[j - 1]], add=True)
        plsc.subcore_barrier()
        pltpu.sync_copy(acc_s.at[pl.ds(s * npt, npt)],
                        out_hbm.at[c, pl.ds(s * npt, npt)])

    return _s(msg, dstp3, zrows)


def _wbuild_body(c_ref, b_ref, o_ref):
    o_ref[0] = jnp.dot(c_ref[0], b_ref[0], preferred_element_type=jnp.float32)


def _wbuild(comps, basis_s):
    return pl.pallas_call(
        _wbuild_body,
        grid=(2, _RP // 80, _NF // 8192),
        in_specs=[
            pl.BlockSpec((1, 80, _B), lambda l, i, j: (l, i, 0)),
            pl.BlockSpec((1, _B, 8192), lambda l, i, j: (l, 0, j)),
        ],
        out_specs=pl.BlockSpec((1, 80, 8192), lambda l, i, j: (l, i, j)),
        out_shape=jax.ShapeDtypeStruct((2, _RP, _NF), jnp.float32),
        compiler_params=pltpu.CompilerParams(
            dimension_semantics=("parallel", "parallel", "parallel")),
    )(comps, basis_s)


def _msg_body(br_ref, xg_ref, w_ref, c_ref, *o_refs):
    cf = c_ref[0, 0, :] + c_ref[0, 1, :]
    full = jnp.dot(xg_ref[...], w_ref[0],
                   preferred_element_type=jnp.float32) * cf[:, None]
    for t, o in enumerate(o_refs):
        o[...] = full[:, t * 128:(t + 1) * 128]


def _msg_call(xg, w3, coef3, block_rel, d_in, d_out):
    nh = d_out // 128
    grid_spec = pltpu.PrefetchScalarGridSpec(
        num_scalar_prefetch=1,
        grid=(_NB,),
        in_specs=[
            pl.BlockSpec((_T, d_in), lambda k, br: (k, 0)),
            pl.BlockSpec((1, d_in, d_out), lambda k, br: (br[k], 0, 0)),
            pl.BlockSpec((1, 2, _T), lambda k, br: (k, 0, 0)),
        ],
        out_specs=[pl.BlockSpec((_T, 128), lambda k, br: (k, 0))] * nh,
    )
    out = pl.pallas_call(
        _msg_body,
        grid_spec=grid_spec,
        out_shape=[jax.ShapeDtypeStruct((_EP, 128), jnp.float32)] * nh,
        compiler_params=pltpu.CompilerParams(dimension_semantics=("arbitrary",)),
    )(block_rel, xg, w3, coef3)
    return out


def _make_combine_body(d_out, relu):
    def body(x_ref, r_ref, b_ref, *a_refs):
        o_ref = a_refs[-1]
        a_refs = a_refs[:-1]
        parts = []
        for t, a in enumerate(a_refs):
            w = min(128, d_out - t * 128)
            parts.append((a[0] + a[1])[:, :w])
        add = parts[0] if len(parts) == 1 else jnp.concatenate(parts, axis=1)
        v = (jnp.dot(x_ref[...], r_ref[...], preferred_element_type=jnp.float32)
             + b_ref[0] + add)
        o_ref[...] = jnp.maximum(v, 0.0) if relu else v
    return body


def _combine_call(x, root, bias, aggs, relu):
    d_in = x.shape[1]
    d_out = root.shape[1]
    return pl.pallas_call(
        _make_combine_body(d_out, relu),
        grid=(_N // 400,),
        in_specs=[
            pl.BlockSpec((400, d_in), lambda k: (k, 0)),
            pl.BlockSpec((d_in, d_out), lambda k: (0, 0)),
            pl.BlockSpec((1, d_out), lambda k: (0, 0)),
        ] + [pl.BlockSpec((2, 400, 128), lambda k: (0, k, 0))] * len(aggs),
        out_specs=pl.BlockSpec((400, d_out), lambda k: (k, 0)),
        out_shape=jax.ShapeDtypeStruct((_N, d_out), jnp.float32),
        compiler_params=pltpu.CompilerParams(dimension_semantics=("arbitrary",)),
    )(x, root, bias.reshape(1, d_out), *aggs)


def kernel(entity_emb, edge_index, edge_type,
           comp1, basis1, root1, bias1,
           comp2, basis2, root2, bias2):
    src = edge_index[0]
    dst = edge_index[1]
    rel = edge_type

    order = jnp.argsort(rel)
    rel_s = rel[order]
    src_s = src[order]
    dst_s = dst[order]
    bounds = jnp.searchsorted(
        rel_s, jnp.arange(_R + 1, dtype=jnp.int32), side='left').astype(jnp.int32)
    starts = bounds[:-1]
    n_r = bounds[1:] - starts
    nb_r = (n_r + _T - 1) // _T
    cb = jnp.cumsum(nb_r).astype(jnp.int32)
    ob = jnp.concatenate([jnp.zeros((1,), jnp.int32), cb[:-1]])
    pos = ob[rel_s] * _T + (jnp.arange(_E, dtype=jnp.int32) - starts[rel_s])
    src_p = jnp.zeros((_EP,), jnp.int32).at[pos].set(src_s)
    dst_p = jnp.zeros((_EP,), jnp.int32).at[pos].set(dst_s)
    key_p = jnp.full((_EP,), _PAD_KEY, jnp.int32).at[pos].set(dst_s * _R + rel_s)
    block_rel = jnp.minimum(
        jnp.searchsorted(cb, jnp.arange(_NB, dtype=jnp.int32), side='right'),
        _R - 1).astype(jnp.int32)

    coef4 = _coef_call(
        key_p.reshape(16, _NCH16, _T),
        jnp.zeros((_ZH,), jnp.float32),
        jnp.zeros((_NCH16, _T), jnp.float32),
    )
    coef3 = jnp.transpose(coef4.reshape(2, _NB, _T), (1, 0, 2))

    comps = jnp.stack([
        jnp.pad(comp1, ((0, _RP - _R), (0, 0))),
        jnp.pad(comp2, ((0, _RP - _R), (0, 0))),
    ])
    basis_s = jnp.stack([
        jnp.pad(basis1, ((0, 0), (0, _D1P - _D1), (0, _D2P - _D2))).reshape(_B, _NF),
        jnp.pad(basis2, ((0, 0), (0, _D2P - _D2), (0, _D1P - _D1))).reshape(_B, _NF),
    ])
    w = _wbuild(comps, basis_s)
    w1 = w[0].reshape(_RP, _D1P, _D2P)
    w2 = w[1].reshape(_RP, _D2P, _D1P)

    srcp3 = src_p.reshape(32, _NCH32, _T)
    dstp3 = dst_p.reshape(32, _NCH32, _T)
    zrows = jnp.zeros((_NP // 16, 128), jnp.float32)

    x_pad = jnp.pad(entity_emb, ((0, 0), (0, _D1P - _D1)))
    root1p = jnp.pad(root1, ((0, 0), (0, _D2P - _D2)))
    bias1p = jnp.pad(bias1, (0, _D2P - _D2))
    root2p = jnp.pad(root2, ((0, _D2P - _D2), (0, 0)))

    xg1 = _gather_call(x_pad, srcp3, _D1P)
    (msg1,) = _msg_call(xg1, w1, coef3, block_rel, _D1P, _D2P)
    agg1 = _scatter_call(msg1, dstp3, zrows, 128)
    h_pad = _combine_call(entity_emb, root1p, bias1p, [agg1], relu=True)

    xg2 = _gather_call(h_pad, srcp3, _D2P)
    msg2a, msg2b = _msg_call(xg2, w2, coef3, block_rel, _D2P, _D1P)
    agg2a = _scatter_call(msg2a, dstp3, zrows, 128)
    agg2b = _scatter_call(msg2b, dstp3, zrows, 128)
    out = _combine_call(h_pad, root2p, bias2, [agg2a, agg2b], relu=False)
    return out


# speedup vs baseline: 12.4165x; 1.2503x over previous
import functools

import jax
import jax.numpy as jnp
from jax import lax
from jax.experimental import pallas as pl
from jax.experimental.pallas import tpu as pltpu
from jax.experimental.pallas import tpu_sc as plsc

_N = 10000
_NP = 10240
_E = 160000
_R = 474
_RP = 480
_B = 30
_D1 = 200
_D2 = 100
_D1P = 256
_D2P = 128
_NF = _D1P * _D2P
_T = 128
_NB = 1728
_EP = _NB * _T
_NCH32 = _EP // (32 * _T)
_NCH16 = _EP // (16 * _T)
_KEYS = _N * _R
_C = 1_185_024
_CS = 1_310_720
_ZH = _CS // 16
_PAD_KEY = 1 << 30

@functools.cache
def _mesh():
    return plsc.VectorSubcoreMesh(core_axis_name="c", subcore_axis_name="s",
                                  num_cores=2, num_subcores=16)


def _coef_call(dst3, relb, zh, zi):
    @functools.partial(
        pl.kernel,
        mesh=_mesh(),
        out_type=jax.ShapeDtypeStruct((2, 16, _NCH16, _T), jnp.float32),
        scratch_types=[
            pltpu.VMEM((_NCH16, _T), jnp.int32),
            pltpu.VMEM((_NCH16, _T), jnp.int32),
            pltpu.VMEM((_NCH16, _T), jnp.float32),
            pltpu.VMEM((_T,), jnp.float32),
            pltpu.VMEM((_T,), jnp.float32),
            pltpu.VMEM((_NCH16,), jnp.int32),
            pltpu.VMEM_SHARED((_CS,), jnp.float32),
        ],
    )
    def _coef_kernel(keys_hbm, relb_hbm, zh_hbm, zi_hbm, coef_hbm,
                     keys_v, idx_v, inv_v, cnt_v, ones_v, relb_v, hist_s):
        c = lax.axis_index("c")
        s = lax.axis_index("s")
        pltpu.sync_copy(keys_hbm.at[s], keys_v)
        pltpu.sync_copy(relb_hbm.at[s], relb_v)
        pltpu.sync_copy(zi_hbm, inv_v)

        def _prep(j, carry):
            r1 = relb_v[pl.ds(j, 1)]
            for k in range(_T // 16):
                sl = pl.ds(k * 16, 16)
                keys_v[j, sl] = keys_v[j, sl] * _R + r1
            return carry

        lax.fori_loop(0, _NCH16, _prep, 0)
        for k in range(_T // 16):
            ones_v[pl.ds(k * 16, 16)] = jnp.ones((16,), jnp.float32)
        for p in range(2):
            lo = (2 * p + c) * _C
            pltpu.sync_copy(zh_hbm, hist_s.at[pl.ds(s * _ZH, _ZH)])
            plsc.subcore_barrier()

            def _scatter(j, carry):
                for k in range(_T // 16):
                    key16 = keys_v[j, pl.ds(k * 16, 16)]
                    d = key16 - lo
                    inr = (d >= 0) & (d < _C)
                    idx_v[j, pl.ds(k * 16, 16)] = jnp.where(inr, d, _C)
                pltpu.sync_copy(ones_v, hist_s.at[idx_v.at[j]], add=True)
                return carry

            lax.fori_loop(0, _NCH16, _scatter, 0)
            plsc.subcore_barrier()

            def _gather(j, carry):
                pltpu.sync_copy(hist_s.at[idx_v.at[j]], cnt_v)
                for k in range(_T // 16):
                    key16 = keys_v[j, pl.ds(k * 16, 16)]
                    d = key16 - lo
                    inr = (d >= 0) & (d < _C)
                    cnt16 = cnt_v[pl.ds(k * 16, 16)]
                    add = jnp.where(inr, 1.0 / cnt16, 0.0)
                    sl = pl.ds(k * 16, 16)
                    inv_v[j, sl] = inv_v[j, sl] + add
                return carry

            lax.fori_loop(0, _NCH16, _gather, 0)
            plsc.subcore_barrier()
        pltpu.sync_copy(inv_v, coef_hbm.at[c, s])


    return _coef_kernel(dst3, relb, zh, zi)

def _gather_call(x, srcp3, d):
    rows_pt = _NCH32 * _T

    @functools.partial(
        pl.kernel,
        mesh=_mesh(),
        out_type=jax.ShapeDtypeStruct((_EP, d), jnp.float32),
        scratch_types=[
            pltpu.VMEM((_NCH32, _T), jnp.int32),
            pltpu.VMEM((_T, d), jnp.float32),
            pltpu.VMEM((_T, d), jnp.float32),
            pltpu.SemaphoreType.DMA,
            pltpu.SemaphoreType.DMA,
        ],
    )
    def _g(x_hbm, srcp_hbm, xg_hbm, idx_v, r0, r1, sem0, sem1):
        c = lax.axis_index("c")
        s = lax.axis_index("s")
        w = s * 2 + c
        base = w * rows_pt
        pltpu.sync_copy(srcp_hbm.at[w], idx_v)
        bufs = (r0, r1)
        sems = (sem0, sem1)
        descs = [None, None]
        descs[0] = pltpu.async_copy(x_hbm.at[idx_v.at[0]], r0, sem0)
        for j in range(1, _NCH32 + 1):
            if j < _NCH32:
                descs[j % 2] = pltpu.async_copy(
                    x_hbm.at[idx_v.at[j]], bufs[j % 2], sems[j % 2])
            descs[(j - 1) % 2].wait()
            pltpu.sync_copy(bufs[(j - 1) % 2],
                            xg_hbm.at[pl.ds(base + (j - 1) * _T, _T)])

    return _g(x, srcp3)


def _scatter_call(msg, dstp3, zrows, d):
    rows_pt = _NCH32 * _T
    npt = _NP // 16

    @functools.partial(
        pl.kernel,
        mesh=_mesh(),
        out_type=jax.ShapeDtypeStruct((2, _NP, d), jnp.float32),
        scratch_types=[
            pltpu.VMEM((_NCH32, _T), jnp.int32),
            pltpu.VMEM((_T, d), jnp.float32),
            pltpu.VMEM((_T, d), jnp.float32),
            pltpu.VMEM_SHARED((_NP, d), jnp.float32),
            pltpu.SemaphoreType.DMA,
            pltpu.SemaphoreType.DMA,
        ],
    )
    def _s(msg_hbm, dstp_hbm, zr_hbm, out_hbm, idx_v, m0, m1, acc_s, sem0, sem1):
        c = lax.axis_index("c")
        s = lax.axis_index("s")
        w = s * 2 + c
        base = w * rows_pt
        pltpu.sync_copy(dstp_hbm.at[w], idx_v)
        pltpu.sync_copy(zr_hbm, acc_s.at[pl.ds(s * npt, npt)])
        plsc.subcore_barrier()
        bufs = (m0, m1)
        sems = (sem0, sem1)
        descs = [None, None]
        descs[0] = pltpu.async_copy(msg_hbm.at[pl.ds(base, _T)], m0, sem0)
        for j in range(1, _NCH32 + 1):
            if j < _NCH32:
                descs[j % 2] = pltpu.async_copy(
                    msg_hbm.at[pl.ds(base + j * _T, _T)], bufs[j % 2], sems[j % 2])
            descs[(j - 1) % 2].wait()
            pltpu.sync_copy(bufs[(j - 1) % 2], acc_s.at[idx_v.at[j - 1]], add=True)
        plsc.subcore_barrier()
        pltpu.sync_copy(acc_s.at[pl.ds(s * npt, npt)],
                        out_hbm.at[c, pl.ds(s * npt, npt)])

    return _s(msg, dstp3, zrows)


def _wbuild_body(c_ref, b_ref, o_ref):
    o_ref[0] = jnp.dot(c_ref[0], b_ref[0], preferred_element_type=jnp.float32)


def _wbuild(comps, basis_s):
    return pl.pallas_call(
        _wbuild_body,
        grid=(2, _RP // 80, _NF // 8192),
        in_specs=[
            pl.BlockSpec((1, 80, _B), lambda l, i, j: (l, i, 0)),
            pl.BlockSpec((1, _B, 8192), lambda l, i, j: (l, 0, j)),
        ],
        out_specs=pl.BlockSpec((1, 80, 8192), lambda l, i, j: (l, i, j)),
        out_shape=jax.ShapeDtypeStruct((2, _RP, _NF), jnp.float32),
        compiler_params=pltpu.CompilerParams(
            dimension_semantics=("parallel", "parallel", "parallel")),
    )(comps, basis_s)


def _msg_body(br_ref, xg_ref, w_ref, c_ref, *o_refs):
    cf = c_ref[0, 0, :] + c_ref[0, 1, :]
    full = jnp.dot(xg_ref[...], w_ref[0],
                   preferred_element_type=jnp.float32) * cf[:, None]
    for t, o in enumerate(o_refs):
        o[...] = full[:, t * 128:(t + 1) * 128]


def _msg_call(xg, w3, coef3, block_rel, d_in, d_out):
    nh = d_out // 128
    grid_spec = pltpu.PrefetchScalarGridSpec(
        num_scalar_prefetch=1,
        grid=(_NB,),
        in_specs=[
            pl.BlockSpec((_T, d_in), lambda k, br: (k, 0)),
            pl.BlockSpec((1, d_in, d_out), lambda k, br: (br[k], 0, 0)),
            pl.BlockSpec((1, 2, _T), lambda k, br: (k, 0, 0)),
        ],
        out_specs=[pl.BlockSpec((_T, 128), lambda k, br: (k, 0))] * nh,
    )
    out = pl.pallas_call(
        _msg_body,
        grid_spec=grid_spec,
        out_shape=[jax.ShapeDtypeStruct((_EP, 128), jnp.float32)] * nh,
        compiler_params=pltpu.CompilerParams(dimension_semantics=("arbitrary",)),
    )(block_rel, xg, w3, coef3)
    return out


def _make_combine_body(d_out, relu):
    def body(x_ref, r_ref, b_ref, *a_refs):
        o_ref = a_refs[-1]
        a_refs = a_refs[:-1]
        parts = []
        for t, a in enumerate(a_refs):
            w = min(128, d_out - t * 128)
            parts.append((a[0] + a[1])[:, :w])
        add = parts[0] if len(parts) == 1 else jnp.concatenate(parts, axis=1)
        v = (jnp.dot(x_ref[...], r_ref[...], preferred_element_type=jnp.float32)
             + b_ref[0] + add)
        o_ref[...] = jnp.maximum(v, 0.0) if relu else v
    return body


def _combine_call(x, root, bias, aggs, relu):
    d_in = x.shape[1]
    d_out = root.shape[1]
    return pl.pallas_call(
        _make_combine_body(d_out, relu),
        grid=(_N // 400,),
        in_specs=[
            pl.BlockSpec((400, d_in), lambda k: (k, 0)),
            pl.BlockSpec((d_in, d_out), lambda k: (0, 0)),
            pl.BlockSpec((1, d_out), lambda k: (0, 0)),
        ] + [pl.BlockSpec((2, 400, 128), lambda k: (0, k, 0))] * len(aggs),
        out_specs=pl.BlockSpec((400, d_out), lambda k: (k, 0)),
        out_shape=jax.ShapeDtypeStruct((_N, d_out), jnp.float32),
        compiler_params=pltpu.CompilerParams(dimension_semantics=("arbitrary",)),
    )(x, root, bias.reshape(1, d_out), *aggs)


def kernel(entity_emb, edge_index, edge_type,
           comp1, basis1, root1, bias1,
           comp2, basis2, root2, bias2):
    src = edge_index[0]
    dst = edge_index[1]
    rel = edge_type

    order = jnp.argsort(rel)
    rel_s = rel[order]
    src_s = src[order]
    dst_s = dst[order]
    i = jnp.arange(_E, dtype=jnp.int32)
    is_start = jnp.concatenate(
        [jnp.ones((1,), jnp.bool_), rel_s[1:] != rel_s[:-1]])
    start_e = lax.cummax(jnp.where(is_start, i, 0), axis=0)
    prev_start = jnp.concatenate([jnp.zeros((1,), jnp.int32), start_e[:-1]])
    pad_jump = jnp.where(is_start, (_T - ((i - prev_start) % _T)) % _T, 0)
    pos = i + jnp.cumsum(pad_jump, dtype=jnp.int32)
    src_p = jnp.zeros((_EP,), jnp.int32).at[pos].set(src_s)
    dst_p = jnp.full((_EP,), _N, jnp.int32).at[pos].set(dst_s)
    bounds = jnp.searchsorted(
        rel_s, jnp.arange(_R + 1, dtype=jnp.int32), side='left').astype(jnp.int32)
    n_r = bounds[1:] - bounds[:-1]
    nb_r = (n_r + _T - 1) // _T
    cb = jnp.cumsum(nb_r).astype(jnp.int32)
    block_rel = jnp.minimum(
        jnp.searchsorted(cb, jnp.arange(_NB, dtype=jnp.int32), side='right'),
        _R - 1).astype(jnp.int32)

    coef4 = _coef_call(
        dst_p.reshape(16, _NCH16, _T),
        block_rel.reshape(16, _NCH16),
        jnp.zeros((_ZH,), jnp.float32),
        jnp.zeros((_NCH16, _T), jnp.float32),
    )
    coef3 = jnp.transpose(coef4.reshape(2, _NB, _T), (1, 0, 2))

    comps = jnp.stack([
        jnp.pad(comp1, ((0, _RP - _R), (0, 0))),
        jnp.pad(comp2, ((0, _RP - _R), (0, 0))),
    ])
    basis_s = jnp.stack([
        jnp.pad(basis1, ((0, 0), (0, _D1P - _D1), (0, _D2P - _D2))).reshape(_B, _NF),
        jnp.pad(basis2, ((0, 0), (0, _D2P - _D2), (0, _D1P - _D1))).reshape(_B, _NF),
    ])
    w = _wbuild(comps, basis_s)
    w1 = w[0].reshape(_RP, _D1P, _D2P)
    w2 = w[1].reshape(_RP, _D2P, _D1P)

    srcp3 = src_p.reshape(32, _NCH32, _T)
    dstp3 = dst_p.reshape(32, _NCH32, _T)
    zrows = jnp.zeros((_NP // 16, 128), jnp.float32)

    x_pad = jnp.pad(entity_emb, ((0, 0), (0, _D1P - _D1)))
    root1p = jnp.pad(root1, ((0, 0), (0, _D2P - _D2)))
    bias1p = jnp.pad(bias1, (0, _D2P - _D2))
    root2p = jnp.pad(root2, ((0, _D2P - _D2), (0, 0)))

    xg1 = _gather_call(x_pad, srcp3, _D1P)
    (msg1,) = _msg_call(xg1, w1, coef3, block_rel, _D1P, _D2P)
    agg1 = _scatter_call(msg1, dstp3, zrows, 128)
    h_pad = _combine_call(entity_emb, root1p, bias1p, [agg1], relu=True)

    xg2 = _gather_call(h_pad, srcp3, _D2P)
    msg2a, msg2b = _msg_call(xg2, w2, coef3, block_rel, _D2P, _D1P)
    agg2a = _scatter_call(msg2a, dstp3, zrows, 128)
    agg2b = _scatter_call(msg2b, dstp3, zrows, 128)
    out = _combine_call(h_pad, root2p, bias2, [agg2a, agg2b], relu=False)
    return out


# speedup vs baseline: 12.4188x; 1.0002x over previous
import functools

import jax
import jax.numpy as jnp
from jax import lax
from jax.experimental import pallas as pl
from jax.experimental.pallas import tpu as pltpu
from jax.experimental.pallas import tpu_sc as plsc

_N = 10000
_NP = 10240
_E = 160000
_R = 474
_RP = 480
_B = 30
_D1 = 200
_D2 = 100
_D1P = 256
_D2P = 128
_NF = _D1P * _D2P
_T = 128
_NB = 1728
_EP = _NB * _T
_NCH32 = _EP // (32 * _T)
_NCH16 = _EP // (16 * _T)
_KEYS = _N * _R
_C = 1_185_024
_CS = 1_310_720
_ZH = _CS // 16
_PAD_KEY = 1 << 30

@functools.cache
def _mesh():
    return plsc.VectorSubcoreMesh(core_axis_name="c", subcore_axis_name="s",
                                  num_cores=2, num_subcores=16)


def _coef_call(dst3, relb, zh, zi):
    @functools.partial(
        pl.kernel,
        mesh=_mesh(),
        out_type=jax.ShapeDtypeStruct((2, 16, _NCH16, _T), jnp.float32),
        scratch_types=[
            pltpu.VMEM((_NCH16, _T), jnp.int32),
            pltpu.VMEM((_NCH16, _T), jnp.int32),
            pltpu.VMEM((_NCH16, _T), jnp.float32),
            pltpu.VMEM((_T,), jnp.float32),
            pltpu.VMEM((_T,), jnp.float32),
            pltpu.VMEM((_NCH16,), jnp.int32),
            pltpu.VMEM_SHARED((_CS,), jnp.float32),
        ],
    )
    def _coef_kernel(keys_hbm, relb_hbm, zh_hbm, zi_hbm, coef_hbm,
                     keys_v, idx_v, inv_v, cnt_v, ones_v, relb_v, hist_s):
        c = lax.axis_index("c")
        s = lax.axis_index("s")
        pltpu.sync_copy(keys_hbm.at[s], keys_v)
        pltpu.sync_copy(relb_hbm.at[s], relb_v)
        pltpu.sync_copy(zi_hbm, inv_v)

        def _prep(j, carry):
            r1 = relb_v[pl.ds(j, 1)]
            for k in range(_T // 16):
                sl = pl.ds(k * 16, 16)
                keys_v[j, sl] = keys_v[j, sl] * _R + r1
            return carry

        lax.fori_loop(0, _NCH16, _prep, 0)
        for k in range(_T // 16):
            ones_v[pl.ds(k * 16, 16)] = jnp.ones((16,), jnp.float32)
        for p in range(2):
            lo = (2 * p + c) * _C
            pltpu.sync_copy(zh_hbm, hist_s.at[pl.ds(s * _ZH, _ZH)])
            plsc.subcore_barrier()

            def _scatter(j, carry):
                for k in range(_T // 16):
                    key16 = keys_v[j, pl.ds(k * 16, 16)]
                    d = key16 - lo
                    inr = (d >= 0) & (d < _C)
                    idx_v[j, pl.ds(k * 16, 16)] = jnp.where(inr, d, _C)
                pltpu.sync_copy(ones_v, hist_s.at[idx_v.at[j]], add=True)
                return carry

            lax.fori_loop(0, _NCH16, _scatter, 0)
            plsc.subcore_barrier()

            def _gather(j, carry):
                pltpu.sync_copy(hist_s.at[idx_v.at[j]], cnt_v)
                for k in range(_T // 16):
                    key16 = keys_v[j, pl.ds(k * 16, 16)]
                    d = key16 - lo
                    inr = (d >= 0) & (d < _C)
                    cnt16 = cnt_v[pl.ds(k * 16, 16)]
                    add = jnp.where(inr, 1.0 / cnt16, 0.0)
                    sl = pl.ds(k * 16, 16)
                    inv_v[j, sl] = inv_v[j, sl] + add
                return carry

            lax.fori_loop(0, _NCH16, _gather, 0)
            plsc.subcore_barrier()
        pltpu.sync_copy(inv_v, coef_hbm.at[c, s])


    return _coef_kernel(dst3, relb, zh, zi)

def _gather_call(x, srcp3, d):
    rows_pt = _NCH32 * _T

    nb = 3

    @functools.partial(
        pl.kernel,
        mesh=_mesh(),
        out_type=jax.ShapeDtypeStruct((_EP, d), jnp.float32),
        scratch_types=[
            pltpu.VMEM((_NCH32, _T), jnp.int32),
        ] + [pltpu.VMEM((_T, d), jnp.float32)] * nb
          + [pltpu.SemaphoreType.DMA] * (2 * nb),
    )
    def _g(x_hbm, srcp_hbm, xg_hbm, idx_v, *bufs_sems):
        bufs = bufs_sems[:nb]
        gsem = bufs_sems[nb:2 * nb]
        wsem = bufs_sems[2 * nb:]
        c = lax.axis_index("c")
        s = lax.axis_index("s")
        w = s * 2 + c
        base = w * rows_pt
        pltpu.sync_copy(srcp_hbm.at[w], idx_v)
        gd = [None] * nb
        wd = [None] * nb
        for j in range(_NCH32):
            b = j % nb
            if j >= nb:
                wd[b].wait()
            gd[b] = pltpu.async_copy(x_hbm.at[idx_v.at[j]], bufs[b], gsem[b])
            if j > 0:
                pb = (j - 1) % nb
                gd[pb].wait()
                wd[pb] = pltpu.async_copy(
                    bufs[pb], xg_hbm.at[pl.ds(base + (j - 1) * _T, _T)],
                    wsem[pb])
        lb = (_NCH32 - 1) % nb
        gd[lb].wait()
        wd[lb] = pltpu.async_copy(
            bufs[lb], xg_hbm.at[pl.ds(base + (_NCH32 - 1) * _T, _T)], wsem[lb])
        for b in range(nb):
            wd[b].wait()

    return _g(x, srcp3)


def _scatter_call(msg, dstp3, zrows, d):
    rows_pt = _NCH32 * _T
    npt = _NP // 16

    @functools.partial(
        pl.kernel,
        mesh=_mesh(),
        out_type=jax.ShapeDtypeStruct((2, _NP, d), jnp.float32),
        scratch_types=[
            pltpu.VMEM((_NCH32, _T), jnp.int32),
            pltpu.VMEM((_T, d), jnp.float32),
            pltpu.VMEM((_T, d), jnp.float32),
            pltpu.VMEM_SHARED((_NP, d), jnp.float32),
            pltpu.SemaphoreType.DMA,
            pltpu.SemaphoreType.DMA,
        ],
    )
    def _s(msg_hbm, dstp_hbm, zr_hbm, out_hbm, idx_v, m0, m1, acc_s, sem0, sem1):
        c = lax.axis_index("c")
        s = lax.axis_index("s")
        w = s * 2 + c
        base = w * rows_pt
        pltpu.sync_copy(dstp_hbm.at[w], idx_v)
        pltpu.sync_copy(zr_hbm, acc_s.at[pl.ds(s * npt, npt)])
        plsc.subcore_barrier()
        bufs = (m0, m1)
        sems = (sem0, sem1)
        descs = [None, None]
        descs[0] = pltpu.async_copy(msg_hbm.at[pl.ds(base, _T)], m0, sem0)
        for j in range(1, _NCH32 + 1):
            if j < _NCH32:
                descs[j % 2] = pltpu.async_copy(
                    msg_hbm.at[pl.ds(base + j * _T, _T)], bufs[j % 2], sems[j % 2])
            descs[(j - 1) % 2].wait()
            pltpu.sync_copy(bufs[(j - 1) % 2], acc_s.at[idx_v.at[j - 1]], add=True)
        plsc.subcore_barrier()
        pltpu.sync_copy(acc_s.at[pl.ds(s * npt, npt)],
                        out_hbm.at[c, pl.ds(s * npt, npt)])

    return _s(msg, dstp3, zrows)


def _wbuild_body(c_ref, b_ref, o_ref):
    o_ref[0] = jnp.dot(c_ref[0], b_ref[0], preferred_element_type=jnp.float32)


def _wbuild(comps, basis_s):
    return pl.pallas_call(
        _wbuild_body,
        grid=(2, _RP // 80, _NF // 8192),
        in_specs=[
            pl.BlockSpec((1, 80, _B), lambda l, i, j: (l, i, 0)),
            pl.BlockSpec((1, _B, 8192), lambda l, i, j: (l, 0, j)),
        ],
        out_specs=pl.BlockSpec((1, 80, 8192), lambda l, i, j: (l, i, j)),
        out_shape=jax.ShapeDtypeStruct((2, _RP, _NF), jnp.float32),
        compiler_params=pltpu.CompilerParams(
            dimension_semantics=("parallel", "parallel", "parallel")),
    )(comps, basis_s)


def _msg_body(br_ref, xg_ref, w_ref, c_ref, *o_refs):
    cf = c_ref[0, 0, :] + c_ref[0, 1, :]
    full = jnp.dot(xg_ref[...], w_ref[0],
                   preferred_element_type=jnp.float32) * cf[:, None]
    for t, o in enumerate(o_refs):
        o[...] = full[:, t * 128:(t + 1) * 128]


def _msg_call(xg, w3, coef3, block_rel, d_in, d_out):
    nh = d_out // 128
    grid_spec = pltpu.PrefetchScalarGridSpec(
        num_scalar_prefetch=1,
        grid=(_NB,),
        in_specs=[
            pl.BlockSpec((_T, d_in), lambda k, br: (k, 0)),
            pl.BlockSpec((1, d_in, d_out), lambda k, br: (br[k], 0, 0)),
            pl.BlockSpec((1, 2, _T), lambda k, br: (k, 0, 0)),
        ],
        out_specs=[pl.BlockSpec((_T, 128), lambda k, br: (k, 0))] * nh,
    )
    out = pl.pallas_call(
        _msg_body,
        grid_spec=grid_spec,
        out_shape=[jax.ShapeDtypeStruct((_EP, 128), jnp.float32)] * nh,
        compiler_params=pltpu.CompilerParams(dimension_semantics=("arbitrary",)),
    )(block_rel, xg, w3, coef3)
    return out


def _make_combine_body(d_out, relu):
    def body(x_ref, r_ref, b_ref, *a_refs):
        o_ref = a_refs[-1]
        a_refs = a_refs[:-1]
        parts = []
        for t, a in enumerate(a_refs):
            w = min(128, d_out - t * 128)
            parts.append((a[0] + a[1])[:, :w])
        add = parts[0] if len(parts) == 1 else jnp.concatenate(parts, axis=1)
        v = (jnp.dot(x_ref[...], r_ref[...], preferred_element_type=jnp.float32)
             + b_ref[0] + add)
        o_ref[...] = jnp.maximum(v, 0.0) if relu else v
    return body


def _combine_call(x, root, bias, aggs, relu):
    d_in = x.shape[1]
    d_out = root.shape[1]
    return pl.pallas_call(
        _make_combine_body(d_out, relu),
        grid=(_N // 400,),
        in_specs=[
            pl.BlockSpec((400, d_in), lambda k: (k, 0)),
            pl.BlockSpec((d_in, d_out), lambda k: (0, 0)),
            pl.BlockSpec((1, d_out), lambda k: (0, 0)),
        ] + [pl.BlockSpec((2, 400, 128), lambda k: (0, k, 0))] * len(aggs),
        out_specs=pl.BlockSpec((400, d_out), lambda k: (k, 0)),
        out_shape=jax.ShapeDtypeStruct((_N, d_out), jnp.float32),
        compiler_params=pltpu.CompilerParams(dimension_semantics=("arbitrary",)),
    )(x, root, bias.reshape(1, d_out), *aggs)


def kernel(entity_emb, edge_index, edge_type,
           comp1, basis1, root1, bias1,
           comp2, basis2, root2, bias2):
    src = edge_index[0]
    dst = edge_index[1]
    rel = edge_type

    order = jnp.argsort(rel)
    rel_s = rel[order]
    src_s = src[order]
    dst_s = dst[order]
    i = jnp.arange(_E, dtype=jnp.int32)
    is_start = jnp.concatenate(
        [jnp.ones((1,), jnp.bool_), rel_s[1:] != rel_s[:-1]])
    start_e = lax.cummax(jnp.where(is_start, i, 0), axis=0)
    prev_start = jnp.concatenate([jnp.zeros((1,), jnp.int32), start_e[:-1]])
    pad_jump = jnp.where(is_start, (_T - ((i - prev_start) % _T)) % _T, 0)
    pos = i + jnp.cumsum(pad_jump, dtype=jnp.int32)
    src_p = jnp.zeros((_EP,), jnp.int32).at[pos].set(src_s)
    dst_p = jnp.full((_EP,), _N, jnp.int32).at[pos].set(dst_s)
    bounds = jnp.searchsorted(
        rel_s, jnp.arange(_R + 1, dtype=jnp.int32), side='left').astype(jnp.int32)
    n_r = bounds[1:] - bounds[:-1]
    nb_r = (n_r + _T - 1) // _T
    cb = jnp.cumsum(nb_r).astype(jnp.int32)
    block_rel = jnp.minimum(
        jnp.searchsorted(cb, jnp.arange(_NB, dtype=jnp.int32), side='right'),
        _R - 1).astype(jnp.int32)

    coef4 = _coef_call(
        dst_p.reshape(16, _NCH16, _T),
        block_rel.reshape(16, _NCH16),
        jnp.zeros((_ZH,), jnp.float32),
        jnp.zeros((_NCH16, _T), jnp.float32),
    )
    coef3 = jnp.transpose(coef4.reshape(2, _NB, _T), (1, 0, 2))

    comps = jnp.stack([
        jnp.pad(comp1, ((0, _RP - _R), (0, 0))),
        jnp.pad(comp2, ((0, _RP - _R), (0, 0))),
    ])
    basis_s = jnp.stack([
        jnp.pad(basis1, ((0, 0), (0, _D1P - _D1), (0, _D2P - _D2))).reshape(_B, _NF),
        jnp.pad(basis2, ((0, 0), (0, _D2P - _D2), (0, _D1P - _D1))).reshape(_B, _NF),
    ])
    w = _wbuild(comps, basis_s)
    w1 = w[0].reshape(_RP, _D1P, _D2P)
    w2 = w[1].reshape(_RP, _D2P, _D1P)

    srcp3 = src_p.reshape(32, _NCH32, _T)
    dstp3 = dst_p.reshape(32, _NCH32, _T)
    zrows = jnp.zeros((_NP // 16, 128), jnp.float32)

    x_pad = jnp.pad(entity_emb, ((0, 0), (0, _D1P - _D1)))
    root1p = jnp.pad(root1, ((0, 0), (0, _D2P - _D2)))
    bias1p = jnp.pad(bias1, (0, _D2P - _D2))
    root2p = jnp.pad(root2, ((0, _D2P - _D2), (0, 0)))

    xg1 = _gather_call(x_pad, srcp3, _D1P)
    (msg1,) = _msg_call(xg1, w1, coef3, block_rel, _D1P, _D2P)
    agg1 = _scatter_call(msg1, dstp3, zrows, 128)
    h_pad = _combine_call(entity_emb, root1p, bias1p, [agg1], relu=True)

    xg2 = _gather_call(h_pad, srcp3, _D2P)
    msg2a, msg2b = _msg_call(xg2, w2, coef3, block_rel, _D2P, _D1P)
    agg2a = _scatter_call(msg2a, dstp3, zrows, 128)
    agg2b = _scatter_call(msg2b, dstp3, zrows, 128)
    out = _combine_call(h_pad, root2p, bias2, [agg2a, agg2b], relu=False)
    return out


# speedup vs baseline: 12.5174x; 1.0079x over previous
import functools

import jax
import jax.numpy as jnp
from jax import lax
from jax.experimental import pallas as pl
from jax.experimental.pallas import tpu as pltpu
from jax.experimental.pallas import tpu_sc as plsc

_N = 10000
_NP = 10240
_E = 160000
_R = 474
_RP = 480
_B = 30
_D1 = 200
_D2 = 100
_D1P = 256
_D2P = 128
_NF = _D1P * _D2P
_T = 128
_NB = 1728
_EP = _NB * _T
_NCH32 = _EP // (32 * _T)
_NCH16 = _EP // (16 * _T)
_KEYS = _N * _R
_C = 1_185_024
_CS = 1_310_720
_ZH = _CS // 16
_PAD_KEY = 1 << 30

@functools.cache
def _mesh():
    return plsc.VectorSubcoreMesh(core_axis_name="c", subcore_axis_name="s",
                                  num_cores=2, num_subcores=16)


def _coef_call(dst3, relb, zh, zi):
    @functools.partial(
        pl.kernel,
        mesh=_mesh(),
        out_type=jax.ShapeDtypeStruct((2, 16, _NCH16, _T), jnp.float32),
        scratch_types=[
            pltpu.VMEM((_NCH16, _T), jnp.int32),
            pltpu.VMEM((_NCH16, _T), jnp.int32),
            pltpu.VMEM((_NCH16, _T), jnp.float32),
            pltpu.VMEM((_T,), jnp.float32),
            pltpu.VMEM((_T,), jnp.float32),
            pltpu.VMEM((_NCH16,), jnp.int32),
            pltpu.VMEM_SHARED((_CS,), jnp.float32),
        ],
    )
    def _coef_kernel(keys_hbm, relb_hbm, zh_hbm, zi_hbm, coef_hbm,
                     keys_v, idx_v, inv_v, cnt_v, ones_v, relb_v, hist_s):
        c = lax.axis_index("c")
        s = lax.axis_index("s")
        pltpu.sync_copy(keys_hbm.at[s], keys_v)
        pltpu.sync_copy(relb_hbm.at[s], relb_v)
        pltpu.sync_copy(zi_hbm, inv_v)

        def _prep(j, carry):
            r1 = relb_v[pl.ds(j, 1)]
            for k in range(_T // 16):
                sl = pl.ds(k * 16, 16)
                keys_v[j, sl] = keys_v[j, sl] * _R + r1
            return carry

        lax.fori_loop(0, _NCH16, _prep, 0)
        for k in range(_T // 16):
            ones_v[pl.ds(k * 16, 16)] = jnp.ones((16,), jnp.float32)
        for p in range(2):
            lo = (2 * p + c) * _C
            pltpu.sync_copy(zh_hbm, hist_s.at[pl.ds(s * _ZH, _ZH)])
            plsc.subcore_barrier()

            def _scatter(j, carry):
                for k in range(_T // 16):
                    key16 = keys_v[j, pl.ds(k * 16, 16)]
                    d = key16 - lo
                    inr = (d >= 0) & (d < _C)
                    idx_v[j, pl.ds(k * 16, 16)] = jnp.where(inr, d, _C)
                pltpu.sync_copy(ones_v, hist_s.at[idx_v.at[j]], add=True)
                return carry

            lax.fori_loop(0, _NCH16, _scatter, 0)
            plsc.subcore_barrier()

            def _gather(j, carry):
                pltpu.sync_copy(hist_s.at[idx_v.at[j]], cnt_v)
                for k in range(_T // 16):
                    key16 = keys_v[j, pl.ds(k * 16, 16)]
                    d = key16 - lo
                    inr = (d >= 0) & (d < _C)
                    cnt16 = cnt_v[pl.ds(k * 16, 16)]
                    add = jnp.where(inr, 1.0 / cnt16, 0.0)
                    sl = pl.ds(k * 16, 16)
                    inv_v[j, sl] = inv_v[j, sl] + add
                return carry

            lax.fori_loop(0, _NCH16, _gather, 0)
            plsc.subcore_barrier()
        pltpu.sync_copy(inv_v, coef_hbm.at[c, s])


    return _coef_kernel(dst3, relb, zh, zi)

def _gather_call(x, srcp3, nrows):
    rows_pt = nrows // 32
    ch = 128
    nch = rows_pt // ch
    nb = 2
    d = 128

    @functools.partial(
        pl.kernel,
        mesh=_mesh(),
        out_type=jax.ShapeDtypeStruct((nrows, d), jnp.float32),
        scratch_types=[
            pltpu.VMEM((nch, ch), jnp.int32),
        ] + [pltpu.VMEM((ch, d), jnp.float32)] * nb
          + [pltpu.SemaphoreType.DMA] * (2 * nb),
    )
    def _g(x_hbm, srcp_hbm, xg_hbm, idx_v, *bufs_sems):
        bufs = bufs_sems[:nb]
        gsem = bufs_sems[nb:2 * nb]
        wsem = bufs_sems[2 * nb:]
        c = lax.axis_index("c")
        s = lax.axis_index("s")
        w = s * 2 + c
        base = w * rows_pt
        pltpu.sync_copy(srcp_hbm.at[w], idx_v)
        gd = [None] * nb
        wd = [None] * nb
        for j in range(nch):
            b = j % nb
            if j >= nb:
                wd[b].wait()
            gd[b] = pltpu.async_copy(x_hbm.at[idx_v.at[j]], bufs[b], gsem[b])
            if j > 0:
                pb = (j - 1) % nb
                gd[pb].wait()
                wd[pb] = pltpu.async_copy(
                    bufs[pb], xg_hbm.at[pl.ds(base + (j - 1) * ch, ch)],
                    wsem[pb])
        lb = (nch - 1) % nb
        gd[lb].wait()
        wd[lb] = pltpu.async_copy(
            bufs[lb], xg_hbm.at[pl.ds(base + (nch - 1) * ch, ch)], wsem[lb])
        for b in range(nb):
            wd[b].wait()

    return _g(x, srcp3.reshape(32, nch, ch))


def _gather_rows(x, src_p, d):
    nh = d // 128
    x2 = x.reshape(x.shape[0] * nh, 128)
    idx = (src_p[:, None] * nh
           + jnp.arange(nh, dtype=jnp.int32)[None, :]).reshape(-1)
    out = _gather_call(x2, idx, _EP * nh)
    return out.reshape(_EP, d)


def _scatter_call(msg, dstp3, zrows, d):
    rows_pt = _NCH32 * _T
    npt = _NP // 16

    @functools.partial(
        pl.kernel,
        mesh=_mesh(),
        out_type=jax.ShapeDtypeStruct((2, _NP, d), jnp.float32),
        scratch_types=[
            pltpu.VMEM((_NCH32, _T), jnp.int32),
            pltpu.VMEM((_T, d), jnp.float32),
            pltpu.VMEM((_T, d), jnp.float32),
            pltpu.VMEM_SHARED((_NP, d), jnp.float32),
            pltpu.SemaphoreType.DMA,
            pltpu.SemaphoreType.DMA,
        ],
    )
    def _s(msg_hbm, dstp_hbm, zr_hbm, out_hbm, idx_v, m0, m1, acc_s, sem0, sem1):
        c = lax.axis_index("c")
        s = lax.axis_index("s")
        w = s * 2 + c
        base = w * rows_pt
        pltpu.sync_copy(dstp_hbm.at[w], idx_v)
        pltpu.sync_copy(zr_hbm, acc_s.at[pl.ds(s * npt, npt)])
        plsc.subcore_barrier()
        bufs = (m0, m1)
        sems = (sem0, sem1)
        descs = [None, None]
        descs[0] = pltpu.async_copy(msg_hbm.at[pl.ds(base, _T)], m0, sem0)
        for j in range(1, _NCH32 + 1):
            if j < _NCH32:
                descs[j % 2] = pltpu.async_copy(
                    msg_hbm.at[pl.ds(base + j * _T, _T)], bufs[j % 2], sems[j % 2])
            descs[(j - 1) % 2].wait()
            pltpu.sync_copy(bufs[(j - 1) % 2], acc_s.at[idx_v.at[j - 1]], add=True)
        plsc.subcore_barrier()
        pltpu.sync_copy(acc_s.at[pl.ds(s * npt, npt)],
                        out_hbm.at[c, pl.ds(s * npt, npt)])

    return _s(msg, dstp3, zrows)


def _wbuild_body(c_ref, b_ref, o_ref):
    o_ref[0] = jnp.dot(c_ref[0], b_ref[0], preferred_element_type=jnp.float32)


def _wbuild(comps, basis_s):
    return pl.pallas_call(
        _wbuild_body,
        grid=(2, _RP // 80, _NF // 8192),
        in_specs=[
            pl.BlockSpec((1, 80, _B), lambda l, i, j: (l, i, 0)),
            pl.BlockSpec((1, _B, 8192), lambda l, i, j: (l, 0, j)),
        ],
        out_specs=pl.BlockSpec((1, 80, 8192), lambda l, i, j: (l, i, j)),
        out_shape=jax.ShapeDtypeStruct((2, _RP, _NF), jnp.float32),
        compiler_params=pltpu.CompilerParams(
            dimension_semantics=("parallel", "parallel", "parallel")),
    )(comps, basis_s)


def _msg_body(br_ref, xg_ref, w_ref, c_ref, *o_refs):
    cf = c_ref[0, 0, :] + c_ref[0, 1, :]
    full = jnp.dot(xg_ref[...], w_ref[0],
                   preferred_element_type=jnp.float32) * cf[:, None]
    for t, o in enumerate(o_refs):
        o[...] = full[:, t * 128:(t + 1) * 128]


def _msg_call(xg, w3, coef3, block_rel, d_in, d_out):
    nh = d_out // 128
    grid_spec = pltpu.PrefetchScalarGridSpec(
        num_scalar_prefetch=1,
        grid=(_NB,),
        in_specs=[
            pl.BlockSpec((_T, d_in), lambda k, br: (k, 0)),
            pl.BlockSpec((1, d_in, d_out), lambda k, br: (br[k], 0, 0)),
            pl.BlockSpec((1, 2, _T), lambda k, br: (k, 0, 0)),
        ],
        out_specs=[pl.BlockSpec((_T, 128), lambda k, br: (k, 0))] * nh,
    )
    out = pl.pallas_call(
        _msg_body,
        grid_spec=grid_spec,
        out_shape=[jax.ShapeDtypeStruct((_EP, 128), jnp.float32)] * nh,
        compiler_params=pltpu.CompilerParams(dimension_semantics=("arbitrary",)),
    )(block_rel, xg, w3, coef3)
    return out


def _make_combine_body(d_out, relu):
    def body(x_ref, r_ref, b_ref, *a_refs):
        o_ref = a_refs[-1]
        a_refs = a_refs[:-1]
        parts = []
        for t, a in enumerate(a_refs):
            w = min(128, d_out - t * 128)
            parts.append((a[0] + a[1])[:, :w])
        add = parts[0] if len(parts) == 1 else jnp.concatenate(parts, axis=1)
        v = (jnp.dot(x_ref[...], r_ref[...], preferred_element_type=jnp.float32)
             + b_ref[0] + add)
        o_ref[...] = jnp.maximum(v, 0.0) if relu else v
    return body


def _combine_call(x, root, bias, aggs, relu):
    d_in = x.shape[1]
    d_out = root.shape[1]
    return pl.pallas_call(
        _make_combine_body(d_out, relu),
        grid=(_N // 400,),
        in_specs=[
            pl.BlockSpec((400, d_in), lambda k: (k, 0)),
            pl.BlockSpec((d_in, d_out), lambda k: (0, 0)),
            pl.BlockSpec((1, d_out), lambda k: (0, 0)),
        ] + [pl.BlockSpec((2, 400, 128), lambda k: (0, k, 0))] * len(aggs),
        out_specs=pl.BlockSpec((400, d_out), lambda k: (k, 0)),
        out_shape=jax.ShapeDtypeStruct((_N, d_out), jnp.float32),
        compiler_params=pltpu.CompilerParams(dimension_semantics=("arbitrary",)),
    )(x, root, bias.reshape(1, d_out), *aggs)


def kernel(entity_emb, edge_index, edge_type,
           comp1, basis1, root1, bias1,
           comp2, basis2, root2, bias2):
    src = edge_index[0]
    dst = edge_index[1]
    rel = edge_type

    order = jnp.argsort(rel)
    rel_s = rel[order]
    src_s = src[order]
    dst_s = dst[order]
    i = jnp.arange(_E, dtype=jnp.int32)
    is_start = jnp.concatenate(
        [jnp.ones((1,), jnp.bool_), rel_s[1:] != rel_s[:-1]])
    start_e = lax.cummax(jnp.where(is_start, i, 0), axis=0)
    prev_start = jnp.concatenate([jnp.zeros((1,), jnp.int32), start_e[:-1]])
    pad_jump = jnp.where(is_start, (_T - ((i - prev_start) % _T)) % _T, 0)
    pos = i + jnp.cumsum(pad_jump, dtype=jnp.int32)
    src_p = jnp.zeros((_EP,), jnp.int32).at[pos].set(src_s)
    dst_p = jnp.full((_EP,), _N, jnp.int32).at[pos].set(dst_s)
    bounds = jnp.searchsorted(
        rel_s, jnp.arange(_R + 1, dtype=jnp.int32), side='left').astype(jnp.int32)
    n_r = bounds[1:] - bounds[:-1]
    nb_r = (n_r + _T - 1) // _T
    cb = jnp.cumsum(nb_r).astype(jnp.int32)
    block_rel = jnp.minimum(
        jnp.searchsorted(cb, jnp.arange(_NB, dtype=jnp.int32), side='right'),
        _R - 1).astype(jnp.int32)

    coef4 = _coef_call(
        dst_p.reshape(16, _NCH16, _T),
        block_rel.reshape(16, _NCH16),
        jnp.zeros((_ZH,), jnp.float32),
        jnp.zeros((_NCH16, _T), jnp.float32),
    )
    coef3 = jnp.transpose(coef4.reshape(2, _NB, _T), (1, 0, 2))

    comps = jnp.stack([
        jnp.pad(comp1, ((0, _RP - _R), (0, 0))),
        jnp.pad(comp2, ((0, _RP - _R), (0, 0))),
    ])
    basis_s = jnp.stack([
        jnp.pad(basis1, ((0, 0), (0, _D1P - _D1), (0, _D2P - _D2))).reshape(_B, _NF),
        jnp.pad(basis2, ((0, 0), (0, _D2P - _D2), (0, _D1P - _D1))).reshape(_B, _NF),
    ])
    w = _wbuild(comps, basis_s)
    w1 = w[0].reshape(_RP, _D1P, _D2P)
    w2 = w[1].reshape(_RP, _D2P, _D1P)

    dstp3 = dst_p.reshape(32, _NCH32, _T)
    zrows = jnp.zeros((_NP // 16, 128), jnp.float32)

    x_pad = jnp.pad(entity_emb, ((0, 0), (0, _D1P - _D1)))
    root1p = jnp.pad(root1, ((0, 0), (0, _D2P - _D2)))
    bias1p = jnp.pad(bias1, (0, _D2P - _D2))
    root2p = jnp.pad(root2, ((0, _D2P - _D2), (0, 0)))

    xg1 = _gather_rows(x_pad, src_p, _D1P)
    (msg1,) = _msg_call(xg1, w1, coef3, block_rel, _D1P, _D2P)
    agg1 = _scatter_call(msg1, dstp3, zrows, 128)
    h_pad = _combine_call(entity_emb, root1p, bias1p, [agg1], relu=True)

    xg2 = _gather_rows(h_pad, src_p, _D2P)
    msg2a, msg2b = _msg_call(xg2, w2, coef3, block_rel, _D2P, _D1P)
    agg2a = _scatter_call(msg2a, dstp3, zrows, 128)
    agg2b = _scatter_call(msg2b, dstp3, zrows, 128)
    out = _combine_call(h_pad, root2p, bias2, [agg2a, agg2b], relu=False)
    return out
